# Initial kernel scaffold; baseline (speedup 1.0000x reference)
import jax
import jax.numpy as jnp
from jax.experimental import pallas as pl


def kernel(g_omics1, features_omics1, g_omics2, features_omics2, W1, att_src, att_dst, W2):
    raise NotImplementedError("write your pallas kernel here")


# speedup vs baseline: 26.1426x; 26.1426x over previous
import functools

import jax
import jax.numpy as jnp
from jax import lax
from jax.experimental import pallas as pl
from jax.experimental.pallas import tpu as pltpu
from jax.experimental.pallas import tpu_sc as plsc

NC, NS, LANES = 2, 16, 16
NW = NC * NS
CHUNK = 80


def _logits_body(x_ref, w1_ref, att2_ref, es_ref, m_ref):
    wcat_t = lax.dot_general(
        att2_ref[...], w1_ref[...],
        dimension_numbers=(((0,), (1,)), ((), ())),
        preferred_element_type=jnp.float32,
    )
    es_t = lax.dot_general(
        wcat_t, x_ref[...],
        dimension_numbers=(((1,), (1,)), ((), ())),
        preferred_element_type=jnp.float32,
    )
    es_ref[...] = es_t
    m = jnp.maximum(jnp.max(es_t[0]) + jnp.max(es_t[1]), 0.0)
    m_ref[...] = jnp.full(m_ref.shape, m, jnp.float32)


def _node_logits(x, w1, att2):
    n = x.shape[0]
    return pl.pallas_call(
        _logits_body,
        out_shape=[
            jax.ShapeDtypeStruct((2, n), jnp.float32),
            jax.ShapeDtypeStruct((8, 128), jnp.float32),
        ],
    )(x, w1, att2)


def _sc_edge_body(src_hbm, dst_hbm, esrc_hbm, edst_hbm, m_hbm, x_hbm,
                  agg_out, den_out,
                  srcv, dstv, pv, rows_v, esrc_v, edst_v, den_v, m_v,
                  agg_s, sem):
    n = esrc_v.shape[0]
    npad = agg_s.shape[0]
    e_total = src_hbm.shape[0]
    epw = e_total // NW
    nchunk = epw // CHUNK
    rows_per_tile = npad // NS

    cid = lax.axis_index("c")
    sid = lax.axis_index("s")
    wid = sid * NC + cid

    pltpu.sync_copy(esrc_hbm, esrc_v)
    pltpu.sync_copy(edst_hbm, edst_v)
    pltpu.sync_copy(m_hbm, m_v)
    mvec = m_v[0, pl.ds(0, LANES)]

    def _zero_den(i, _):
        den_v[pl.ds(i * LANES, LANES)] = jnp.zeros((LANES,), jnp.float32)
        return _
    lax.fori_loop(0, n // LANES, _zero_den, None)

    def _zero_rows(j, _):
        for c in range(128 // LANES):
            rows_v[j, pl.ds(c * LANES, LANES)] = jnp.zeros((LANES,), jnp.float32)
        return _
    lax.fori_loop(0, CHUNK, _zero_rows, None)

    for b in range(rows_per_tile // CHUNK):
        pltpu.sync_copy(rows_v, agg_s.at[pl.ds(sid * rows_per_tile + b * CHUNK, CHUNK)])
    plsc.subcore_barrier()

    def _chunk(i, _):
        off = wid * epw + i * CHUNK
        pltpu.sync_copy(src_hbm.at[pl.ds(off, CHUNK)], srcv)
        pltpu.sync_copy(dst_hbm.at[pl.ds(off, CHUNK)], dstv)

        for g in range(CHUNK // LANES):
            i_s = srcv[pl.ds(g * LANES, LANES)]
            i_d = dstv[pl.ds(g * LANES, LANES)]
            u = plsc.load_gather(esrc_v, [i_s]) + plsc.load_gather(edst_v, [i_d])
            p = jnp.exp(jnp.maximum(u, 0.2 * u) - mvec)
            pv[pl.ds(g * LANES, LANES)] = p
            plsc.addupdate_scatter(den_v, [i_d], p)

        pltpu.async_copy(x_hbm.at[srcv], rows_v, sem).wait()

        def _scale(gg, _):
            sv = pv[pl.ds(gg * LANES, LANES)]
            for j in range(LANES):
                s = sv[j]
                row = gg * LANES + j
                for c in range(128 // LANES):
                    sl = pl.ds(c * LANES, LANES)
                    rows_v[row, sl] = rows_v[row, sl] * s
            return _
        lax.fori_loop(0, CHUNK // LANES, _scale, None)

        pltpu.sync_copy(rows_v, agg_s.at[dstv], add=True)
        return _

    lax.fori_loop(0, nchunk, _chunk, None)
    plsc.subcore_barrier()

    pltpu.sync_copy(den_v, den_out.at[wid])
    row0 = sid * rows_per_tile
    pltpu.sync_copy(agg_s.at[pl.ds(row0, rows_per_tile)],
                    agg_out.at[cid, pl.ds(row0, rows_per_tile)])


def _sc_edge_aggregate(src, dst, esrc, edst, m8, x):
    n, in_dim = x.shape
    npad = ((n + NW * CHUNK - 1) // (NW * CHUNK)) * NW * CHUNK
    mesh = plsc.VectorSubcoreMesh(
        core_axis_name="c", subcore_axis_name="s",
        num_cores=NC, num_subcores=NS,
    )
    run = pl.kernel(
        _sc_edge_body,
        out_type=[
            jax.ShapeDtypeStruct((NC, npad, in_dim), jnp.float32),
            jax.ShapeDtypeStruct((NW, n), jnp.float32),
        ],
        mesh=mesh,
        compiler_params=pltpu.CompilerParams(needs_layout_passes=False),
        scratch_types=[
            pltpu.VMEM((CHUNK,), jnp.int32),
            pltpu.VMEM((CHUNK,), jnp.int32),
            pltpu.VMEM((CHUNK,), jnp.float32),
            pltpu.VMEM((CHUNK, in_dim), jnp.float32),
            pltpu.VMEM((n,), jnp.float32),
            pltpu.VMEM((n,), jnp.float32),
            pltpu.VMEM((n,), jnp.float32),
            pltpu.VMEM((8, 128), jnp.float32),
            pltpu.VMEM_SHARED((npad, in_dim), jnp.float32),
            pltpu.SemaphoreType.DMA,
        ],
    )
    return run(src, dst, esrc, edst, m8, x)


def _tail_body(agg0_ref, agg1_ref, den_ref, w1_ref, w2_ref, out_ref):
    agg = agg0_ref[...] + agg1_ref[...]
    den = jnp.sum(den_ref[...], axis=1)
    a = agg / (den + 1e-16)[:, None]
    h = jnp.dot(a, w1_ref[...], preferred_element_type=jnp.float32)
    h1 = jnp.where(h > 0, h, jnp.exp(h) - 1.0)
    o = jnp.dot(h1, w2_ref[...], preferred_element_type=jnp.float32)
    nrm = jnp.sqrt(jnp.sum(o * o, axis=-1, keepdims=True))
    out_ref[...] = o / (nrm + 1e-12)


def _tail(agg0, agg1, den, w1, w2):
    n, in_dim = agg0.shape
    out_dim = w2.shape[1]
    blk = 1000
    grid = n // blk
    return pl.pallas_call(
        _tail_body,
        grid=(grid,),
        in_specs=[
            pl.BlockSpec((blk, in_dim), lambda i: (i, 0)),
            pl.BlockSpec((blk, in_dim), lambda i: (i, 0)),
            pl.BlockSpec((blk, NW), lambda i: (i, 0)),
            pl.BlockSpec(w1.shape, lambda i: (0, 0)),
            pl.BlockSpec(w2.shape, lambda i: (0, 0)),
        ],
        out_specs=pl.BlockSpec((blk, out_dim), lambda i: (i, 0)),
        out_shape=jax.ShapeDtypeStruct((n, out_dim), jnp.float32),
    )(agg0, agg1, den, w1, w2)


def _encode(g, x, w1, att2, w2):
    n = x.shape[0]
    es, m8 = _node_logits(x, w1, att2)
    aggp, denp = _sc_edge_aggregate(
        g[0], g[1], es[0], es[1], m8, x)
    return _tail(aggp[0, :n], aggp[1, :n], denp.T, w1, w2)


@jax.jit
def kernel(g_omics1, features_omics1, g_omics2, features_omics2,
           W1, att_src, att_dst, W2):
    att2 = jnp.stack([att_src, att_dst], axis=1)
    z1 = _encode(g_omics1, features_omics1, W1, att2, W2)
    z2 = _encode(g_omics2, features_omics2, W1, att2, W2)
    return (z1, z2)
